# Initial kernel scaffold; baseline (speedup 1.0000x reference)
import jax
import jax.numpy as jnp
from jax.experimental import pallas as pl


def kernel(x, edge_index, Wq1, bq1, Wk1, bk1, Wv1, bv1, Ws1, bs1, Wq2, bq2, Wk2, bk2, Wv2, bv2, Ws2, bs2):
    raise NotImplementedError("write your pallas kernel here")


# speedup vs baseline: 1.1048x; 1.1048x over previous
import functools

import jax
import jax.numpy as jnp
from jax.experimental import pallas as pl
from jax.experimental.pallas import tpu as pltpu

N = 10000
E = 320000


def _mm_kernel(x_ref, w_ref, b_ref, o_ref):
    o_ref[...] = jnp.dot(x_ref[...], w_ref[...],
                         preferred_element_type=jnp.float32) + b_ref[...]


def _mm(x, w, b, blk=1000):
    n, kdim = x.shape
    m = w.shape[1]
    return pl.pallas_call(
        _mm_kernel,
        grid=(n // blk,),
        in_specs=[
            pl.BlockSpec((blk, kdim), lambda i: (i, 0)),
            pl.BlockSpec((kdim, m), lambda i: (0, 0)),
            pl.BlockSpec((1, m), lambda i: (0, 0)),
        ],
        out_specs=pl.BlockSpec((blk, m), lambda i: (i, 0)),
        out_shape=jax.ShapeDtypeStruct((n, m), jnp.float32),
    )(x, w, b.reshape(1, -1))


def _tconv(x, edge_index, Wq, bq, Wk, bk, Wv, bv, Ws, bs, d_out):
    src = edge_index[0]
    dst = edge_index[1]
    n = x.shape[0]
    q = _mm(x, Wq, bq)
    k = _mm(x, Wk, bk)
    v = _mm(x, Wv, bv)
    alpha = jnp.sum(q[dst] * k[src], axis=-1) / jnp.sqrt(jnp.float32(d_out))
    amax = jax.ops.segment_max(alpha, dst, num_segments=n)
    amax = jnp.where(jnp.isfinite(amax), amax, 0.0)
    ex = jnp.exp(alpha - amax[dst])
    denom = jax.ops.segment_sum(ex, dst, num_segments=n)
    w = ex / (denom[dst] + 1e-16)
    out = jax.ops.segment_sum(w[:, None] * v[src], dst, num_segments=n)
    return out + _mm(x, Ws, bs)


def kernel(x, edge_index, Wq1, bq1, Wk1, bk1, Wv1, bv1, Ws1, bs1,
           Wq2, bq2, Wk2, bk2, Wv2, bv2, Ws2, bs2):
    h1 = jax.nn.elu(_tconv(x, edge_index, Wq1, bq1, Wk1, bk1, Wv1, bv1,
                           Ws1, bs1, 512))
    h2 = jax.nn.elu(_tconv(h1, edge_index, Wq2, bq2, Wk2, bk2, Wv2, bv2,
                           Ws2, bs2, 64))
    return h2


# speedup vs baseline: 5.4105x; 4.8975x over previous
import functools

import jax
import jax.numpy as jnp
from jax import lax
from jax.experimental import pallas as pl
from jax.experimental.pallas import tpu as pltpu
from jax.experimental.pallas import tpu_sc as plsc

N = 10000
E = 320000
NC = 2
NS = 16
NW = NC * NS
B = 80
NBATCH = E // B
RPT = E // NW // B
D1 = 512
D2 = 64
D1P = 144


def _elu(x):
    return jnp.where(x > 0, x, jnp.exp(jnp.minimum(x, 0.0)) - 1.0)


def _mmt_kernel(a_ref, b_ref, o_ref, *, scale):
    o_ref[...] = lax.dot_general(
        a_ref[...], b_ref[...], (((1,), (1,)), ((), ())),
        preferred_element_type=jnp.float32) * scale


def _mmt(a, b, scale):
    m, n = a.shape[0], b.shape[0]
    return pl.pallas_call(
        functools.partial(_mmt_kernel, scale=scale),
        out_shape=jax.ShapeDtypeStruct((m, n), jnp.float32),
    )(a, b)


def _rowmm_kernel(x_ref, w_ref, o_ref):
    o_ref[...] = jnp.dot(x_ref[...], w_ref[...],
                         preferred_element_type=jnp.float32)


def _rowmm(x, w, blk=1000):
    n, k = x.shape
    m = w.shape[1]
    return pl.pallas_call(
        _rowmm_kernel,
        grid=(n // blk,),
        in_specs=[pl.BlockSpec((blk, k), lambda i: (i, 0)),
                  pl.BlockSpec((k, m), lambda i: (0, 0))],
        out_specs=pl.BlockSpec((blk, m), lambda i: (i, 0)),
        out_shape=jax.ShapeDtypeStruct((n, m), jnp.float32),
    )(x, w)


def _proj1_kernel(x_ref, w_ref, b_ref, *out_refs):
    h = jnp.dot(x_ref[...], w_ref[...],
                preferred_element_type=jnp.float32) + b_ref[...]
    for i in range(8):
        out_refs[i][...] = h[:, 64 * i:64 * (i + 1)]
    out_refs[8][...] = h[:, 512:1024]


def _proj1(x, wvs, bvs, blk=1000):
    vshape = jax.ShapeDtypeStruct((N, 64), jnp.float32)
    return pl.pallas_call(
        _proj1_kernel,
        grid=(N // blk,),
        in_specs=[pl.BlockSpec((blk, 128), lambda i: (i, 0)),
                  pl.BlockSpec((128, 1024), lambda i: (0, 0)),
                  pl.BlockSpec((1, 1024), lambda i: (0, 0))],
        out_specs=[pl.BlockSpec((blk, 64), lambda i: (i, 0))] * 8
        + [pl.BlockSpec((blk, 512), lambda i: (i, 0))],
        out_shape=[vshape] * 8 + [jax.ShapeDtypeStruct((N, 512), jnp.float32)],
    )(x, wvs, bvs)


def _densum_kernel(d_ref, o_ref):
    o_ref[...] = jnp.sum(d_ref[...], axis=0, keepdims=True)


def _densum(den):
    return pl.pallas_call(
        _densum_kernel,
        out_shape=jax.ShapeDtypeStruct((1, N), jnp.float32),
    )(den)


def _out1proj2_kernel(*refs):
    ocs = refs[:8]
    s_ref, w_ref, b_ref, q2, k2, v2, s2 = refs[8:]
    h1 = jnp.concatenate(
        [o[0] + o[1] for o in ocs], axis=1) + s_ref[...]
    h1 = _elu(h1)
    hh = jnp.dot(h1, w_ref[...], preferred_element_type=jnp.float32) \
        + b_ref[...]
    q2[...] = hh[:, 0:64] * 0.125
    k2[...] = hh[:, 64:128]
    v2[...] = hh[:, 128:192]
    s2[...] = hh[:, 192:256]


def _out1proj2(oc, s1, w2, b2, blk=1000):
    oshape = jax.ShapeDtypeStruct((N, 64), jnp.float32)
    return pl.pallas_call(
        _out1proj2_kernel,
        grid=(N // blk,),
        in_specs=[pl.BlockSpec((NC, blk, 64), lambda i: (0, i, 0))] * 8
        + [pl.BlockSpec((blk, 512), lambda i: (i, 0)),
           pl.BlockSpec((512, 256), lambda i: (0, 0)),
           pl.BlockSpec((1, 256), lambda i: (0, 0))],
        out_specs=[pl.BlockSpec((blk, 64), lambda i: (i, 0))] * 4,
        out_shape=[oshape] * 4,
    )(*oc, s1, w2, b2)


def _out2_kernel(o_ref, s_ref, h_ref):
    h_ref[...] = _elu(o_ref[0] + o_ref[1] + s_ref[...])


def _out2(o2, s2, blk=1000):
    return pl.pallas_call(
        _out2_kernel,
        grid=(N // blk,),
        in_specs=[pl.BlockSpec((NC, blk, 64), lambda i: (0, i, 0)),
                  pl.BlockSpec((blk, 64), lambda i: (i, 0))],
        out_specs=pl.BlockSpec((blk, 64), lambda i: (i, 0)),
        out_shape=jax.ShapeDtypeStruct((N, 64), jnp.float32),
    )(o2, s2)


_MESH = plsc.VectorSubcoreMesh(core_axis_name="c", subcore_axis_name="s",
                               num_cores=NC, num_subcores=NS)


def _edge_body(D, p_hbm, k_hbm, src_hbm, dst_hbm, ex_hbm, den_hbm,
               src_v, dst_v, ex_v, den_v, qb, kb, sq, sk):
    c = lax.axis_index("c")
    s = lax.axis_index("s")
    wid = c * NS + s
    pltpu.sync_copy(src_hbm.at[wid], src_v)
    pltpu.sync_copy(dst_hbm.at[wid], dst_v)

    zero16 = jnp.zeros((16,), jnp.float32)

    def zden(i, carry):
        den_v[pl.ds(i * 16, 16)] = zero16
        return carry

    lax.fori_loop(0, N // 16, zden, 0)

    iota = lax.iota(jnp.int32, 16)

    def batch(b, carry):
        cq = pltpu.async_copy(p_hbm.at[dst_v.at[b]], qb, sq)
        ck = pltpu.async_copy(k_hbm.at[src_v.at[b]], kb, sk)
        cq.wait()
        ck.wait()
        for g in range(B // 16):
            row16 = iota + g * 16
            d16 = dst_v[b, pl.ds(g * 16, 16)]

            def dotstep(j16, acc):
                for jj in range(16):
                    col = jnp.full((16,), j16 * 16 + jj, jnp.int32)
                    acc = acc + (plsc.load_gather(qb, [row16, col])
                                 * plsc.load_gather(kb, [row16, col]))
                return acc

            acc = lax.fori_loop(0, D // 16, dotstep, zero16)
            ex16 = jnp.exp(acc)
            ex_v[b, pl.ds(g * 16, 16)] = ex16
            plsc.addupdate_scatter(den_v, [d16], ex16)
        return carry

    lax.fori_loop(0, RPT, batch, 0)
    pltpu.sync_copy(ex_v, ex_hbm.at[wid])
    pltpu.sync_copy(den_v, den_hbm.at[pl.ds(wid * N, N)])


def _make_edge(D):
    return pl.kernel(
        functools.partial(_edge_body, D),
        out_type=[jax.ShapeDtypeStruct((NW, RPT, B), jnp.float32),
                  jax.ShapeDtypeStruct((NW * N,), jnp.float32)],
        mesh=_MESH,
        compiler_params=pltpu.CompilerParams(use_tc_tiling_on_sc=False, needs_layout_passes=False),
        scratch_types=[
            pltpu.VMEM((RPT, B), jnp.int32),
            pltpu.VMEM((RPT, B), jnp.int32),
            pltpu.VMEM((RPT, B), jnp.float32),
            pltpu.VMEM((N,), jnp.float32),
            pltpu.VMEM((B, D), jnp.float32),
            pltpu.VMEM((B, D), jnp.float32),
            pltpu.SemaphoreType.DMA,
            pltpu.SemaphoreType.DMA,
        ],
    )


def _agg_body(Dc, v_hbm, src_hbm, dst_hbm, ex_hbm, ds_hbm, z_hbm, o_hbm,
              src_v, dst_v, w_v, den_v, vb0, vb1, s0, s1, spm):
    c = lax.axis_index("c")
    s = lax.axis_index("s")
    wid = c * NS + s

    @pl.when(s == 0)
    def _():
        pltpu.sync_copy(z_hbm, spm)
    plsc.subcore_barrier()

    pltpu.sync_copy(ds_hbm.at[0], den_v)
    pltpu.sync_copy(src_hbm.at[wid], src_v)
    pltpu.sync_copy(dst_hbm.at[wid], dst_v)
    pltpu.sync_copy(ex_hbm.at[wid], w_v)

    def wstep(b, carry):
        for g in range(B // 16):
            d16 = dst_v[b, pl.ds(g * 16, 16)]
            dv = plsc.load_gather(den_v, [d16])
            w_v[b, pl.ds(g * 16, 16)] = \
                w_v[b, pl.ds(g * 16, 16)] / (dv + 1e-16)
        return carry

    lax.fori_loop(0, RPT, wstep, 0)

    def process(b, vb):
        brow = jnp.full((16,), b, jnp.int32)
        for g in range(B // 16):
            for e in range(16):
                col = jnp.full((16,), g * 16 + e, jnp.int32)
                sp = plsc.load_gather(w_v, [brow, col])
                for j in range(Dc // 16):
                    vb[g * 16 + e, pl.ds(j * 16, 16)] = \
                        vb[g * 16 + e, pl.ds(j * 16, 16)] * sp
        pltpu.sync_copy(vb, spm.at[dst_v.at[b]], add=True)

    pltpu.async_copy(v_hbm.at[src_v.at[0]], vb0, s0)

    def loop(i, carry):
        b0 = 2 * i
        pltpu.async_copy(v_hbm.at[src_v.at[b0 + 1]], vb1, s1)
        pltpu.make_async_copy(v_hbm.at[src_v.at[b0]], vb0, s0).wait()
        process(b0, vb0)
        pltpu.async_copy(v_hbm.at[src_v.at[b0 + 2]], vb0, s0)
        pltpu.make_async_copy(v_hbm.at[src_v.at[b0 + 1]], vb1, s1).wait()
        process(b0 + 1, vb1)
        return carry

    lax.fori_loop(0, (RPT - 1) // 2, loop, 0)
    pltpu.make_async_copy(v_hbm.at[src_v.at[RPT - 1]], vb0, s0).wait()
    process(RPT - 1, vb0)

    plsc.subcore_barrier()

    @pl.when(s == 0)
    def _():
        pltpu.sync_copy(spm, o_hbm.at[c])


def _make_agg(Dc):
    return pl.kernel(
        functools.partial(_agg_body, Dc),
        out_type=jax.ShapeDtypeStruct((NC, N, Dc), jnp.float32),
        mesh=_MESH,
        compiler_params=pltpu.CompilerParams(use_tc_tiling_on_sc=False, needs_layout_passes=False),
        scratch_types=[
            pltpu.VMEM((RPT, B), jnp.int32),
            pltpu.VMEM((RPT, B), jnp.int32),
            pltpu.VMEM((RPT, B), jnp.float32),
            pltpu.VMEM((N,), jnp.float32),
            pltpu.VMEM((B, Dc), jnp.float32),
            pltpu.VMEM((B, Dc), jnp.float32),
            pltpu.SemaphoreType.DMA,
            pltpu.SemaphoreType.DMA,
            pltpu.VMEM_SHARED((N, Dc), jnp.float32),
        ],
    )


_edge144 = _make_edge(D1P)
_edge64 = _make_edge(D2)
_agg64 = _make_agg(D2)


def kernel(x, edge_index, Wq1, bq1, Wk1, bk1, Wv1, bv1, Ws1, bs1,
           Wq2, bq2, Wk2, bk2, Wv2, bv2, Ws2, bs2):
    f32 = jnp.float32
    xp = jnp.concatenate(
        [x, jnp.ones((N, 1), f32), jnp.zeros((N, D1P - 129), f32)], axis=1)
    wq1p = jnp.concatenate(
        [Wq1, bq1[None, :], jnp.zeros((D1P - 129, D1), f32)], axis=0)
    wk1p = jnp.concatenate(
        [Wk1, bk1[None, :], jnp.zeros((D1P - 129, D1), f32)], axis=0)

    m1 = _mmt(wq1p, wk1p, 1.0 / (D1 ** 0.5))
    p1 = _rowmm(xp, m1)

    wvs1 = jnp.concatenate([Wv1, Ws1], axis=1)
    bvs1 = jnp.concatenate([bv1, bs1])[None, :]
    *v1c, s1 = _proj1(x, wvs1, bvs1)

    src2 = edge_index[0].reshape(NW, RPT, B)
    dst2 = edge_index[1].reshape(NW, RPT, B)
    z64 = jnp.zeros((N, D2), f32)

    ex1, den1 = _edge144(p1, xp, src2, dst2)
    ds1 = _densum(den1.reshape(NW, N))
    o1 = [_agg64(v, src2, dst2, ex1, ds1, z64) for v in v1c]

    w2 = jnp.concatenate([Wq2, Wk2, Wv2, Ws2], axis=1)
    b2 = jnp.concatenate([bq2, bk2, bv2, bs2])[None, :]
    q2s, k2, v2, s2 = _out1proj2(o1, s1, w2, b2)

    ex2, den2 = _edge64(q2s, k2, src2, dst2)
    ds2 = _densum(den2.reshape(NW, N))
    o2 = _agg64(v2, src2, dst2, ex2, ds2, z64)
    return _out2(o2, s2)


# speedup vs baseline: 5.8733x; 1.0855x over previous
import functools

import jax
import jax.numpy as jnp
from jax import lax
from jax.experimental import pallas as pl
from jax.experimental.pallas import tpu as pltpu
from jax.experimental.pallas import tpu_sc as plsc

N = 10000
E = 320000
NC = 2
NS = 16
NW = NC * NS
B = 100
RPT = E // NW // B
D1 = 512
D2 = 64
D1P = 144


def _elu(x):
    return jnp.where(x > 0, x, jnp.exp(jnp.minimum(x, 0.0)) - 1.0)


def _mmt_kernel(a_ref, b_ref, o_ref, *, scale):
    o_ref[...] = lax.dot_general(
        a_ref[...], b_ref[...], (((1,), (1,)), ((), ())),
        preferred_element_type=jnp.float32) * scale


def _mmt(a, b, scale):
    m, n = a.shape[0], b.shape[0]
    return pl.pallas_call(
        functools.partial(_mmt_kernel, scale=scale),
        out_shape=jax.ShapeDtypeStruct((m, n), jnp.float32),
    )(a, b)


def _rowmm_kernel(x_ref, w_ref, o_ref):
    o_ref[...] = jnp.dot(x_ref[...], w_ref[...],
                         preferred_element_type=jnp.float32)


def _rowmm(x, w, blk=1000):
    n, k = x.shape
    m = w.shape[1]
    return pl.pallas_call(
        _rowmm_kernel,
        grid=(n // blk,),
        in_specs=[pl.BlockSpec((blk, k), lambda i: (i, 0)),
                  pl.BlockSpec((k, m), lambda i: (0, 0))],
        out_specs=pl.BlockSpec((blk, m), lambda i: (i, 0)),
        out_shape=jax.ShapeDtypeStruct((n, m), jnp.float32),
    )(x, w)


def _proj1_kernel(x_ref, w_ref, b_ref, *out_refs):
    h = jnp.dot(x_ref[...], w_ref[...],
                preferred_element_type=jnp.float32) + b_ref[...]
    for i in range(8):
        out_refs[i][...] = h[:, 64 * i:64 * (i + 1)]
    out_refs[8][...] = h[:, 512:1024]


def _proj1(x, wvs, bvs, blk=1000):
    vshape = jax.ShapeDtypeStruct((N, 64), jnp.float32)
    return pl.pallas_call(
        _proj1_kernel,
        grid=(N // blk,),
        in_specs=[pl.BlockSpec((blk, 128), lambda i: (i, 0)),
                  pl.BlockSpec((128, 1024), lambda i: (0, 0)),
                  pl.BlockSpec((1, 1024), lambda i: (0, 0))],
        out_specs=[pl.BlockSpec((blk, 64), lambda i: (i, 0))] * 8
        + [pl.BlockSpec((blk, 512), lambda i: (i, 0))],
        out_shape=[vshape] * 8 + [jax.ShapeDtypeStruct((N, 512), jnp.float32)],
    )(x, wvs, bvs)


def _densum_kernel(d_ref, o_ref):
    o_ref[...] = jnp.sum(d_ref[...], axis=0, keepdims=True)


def _densum(den):
    return pl.pallas_call(
        _densum_kernel,
        out_shape=jax.ShapeDtypeStruct((1, N), jnp.float32),
    )(den)


def _out1proj2_kernel(*refs):
    ocs = refs[:8]
    s_ref, w_ref, b_ref, q2, k2, v2, s2 = refs[8:]
    h1 = jnp.concatenate(
        [o[0] + o[1] for o in ocs], axis=1) + s_ref[...]
    h1 = _elu(h1)
    hh = jnp.dot(h1, w_ref[...], preferred_element_type=jnp.float32) \
        + b_ref[...]
    q2[...] = hh[:, 0:64] * 0.125
    k2[...] = hh[:, 64:128]
    v2[...] = hh[:, 128:192]
    s2[...] = hh[:, 192:256]


def _out1proj2(oc, s1, w2, b2, blk=1000):
    oshape = jax.ShapeDtypeStruct((N, 64), jnp.float32)
    return pl.pallas_call(
        _out1proj2_kernel,
        grid=(N // blk,),
        in_specs=[pl.BlockSpec((NC, blk, 64), lambda i: (0, i, 0))] * 8
        + [pl.BlockSpec((blk, 512), lambda i: (i, 0)),
           pl.BlockSpec((512, 256), lambda i: (0, 0)),
           pl.BlockSpec((1, 256), lambda i: (0, 0))],
        out_specs=[pl.BlockSpec((blk, 64), lambda i: (i, 0))] * 4,
        out_shape=[oshape] * 4,
    )(*oc, s1, w2, b2)


def _out2_kernel(o_ref, s_ref, h_ref):
    h_ref[...] = _elu(o_ref[0] + o_ref[1] + s_ref[...])


def _out2(o2, s2, blk=1000):
    return pl.pallas_call(
        _out2_kernel,
        grid=(N // blk,),
        in_specs=[pl.BlockSpec((NC, blk, 64), lambda i: (0, i, 0)),
                  pl.BlockSpec((blk, 64), lambda i: (i, 0))],
        out_specs=pl.BlockSpec((blk, 64), lambda i: (i, 0)),
        out_shape=jax.ShapeDtypeStruct((N, 64), jnp.float32),
    )(o2, s2)


_MESH = plsc.VectorSubcoreMesh(core_axis_name="c", subcore_axis_name="s",
                               num_cores=NC, num_subcores=NS)


def _edge_body(D, p_hbm, k_hbm, src_hbm, dst_hbm, ex_hbm, den_hbm,
               src_v, dst_v, ex_v, den_v, qb0, kb0, qb1, kb1,
               sq0, sk0, sq1, sk1):
    c = lax.axis_index("c")
    s = lax.axis_index("s")
    wid = c * NS + s
    pltpu.sync_copy(src_hbm.at[wid], src_v)
    pltpu.sync_copy(dst_hbm.at[wid], dst_v)

    zero16 = jnp.zeros((16,), jnp.float32)

    def zden(i, carry):
        den_v[pl.ds(i * 16, 16)] = zero16
        return carry

    lax.fori_loop(0, N // 16, zden, 0)

    iota = lax.iota(jnp.int32, 16)

    def process(b, qb, kb):
        for g in range(B // 16):
            row16 = iota + g * 16
            d16 = dst_v[b, pl.ds(g * 16, 16)]

            def dotstep(j16, acc):
                for jj in range(16):
                    col = jnp.full((16,), j16 * 16 + jj, jnp.int32)
                    acc = acc + (plsc.load_gather(qb, [row16, col])
                                 * plsc.load_gather(kb, [row16, col]))
                return acc

            acc = lax.fori_loop(0, D // 16, dotstep, zero16)
            ex16 = jnp.exp(acc)
            ex_v[b, pl.ds(g * 16, 16)] = ex16
            plsc.addupdate_scatter(den_v, [d16], ex16)

    pltpu.async_copy(p_hbm.at[dst_v.at[0]], qb0, sq0)
    pltpu.async_copy(k_hbm.at[src_v.at[0]], kb0, sk0)

    def loop(i, carry):
        b0 = 2 * i
        pltpu.async_copy(p_hbm.at[dst_v.at[b0 + 1]], qb1, sq1)
        pltpu.async_copy(k_hbm.at[src_v.at[b0 + 1]], kb1, sk1)
        pltpu.make_async_copy(p_hbm.at[dst_v.at[b0]], qb0, sq0).wait()
        pltpu.make_async_copy(k_hbm.at[src_v.at[b0]], kb0, sk0).wait()
        process(b0, qb0, kb0)

        @pl.when(i < RPT // 2 - 1)
        def _():
            pltpu.async_copy(p_hbm.at[dst_v.at[b0 + 2]], qb0, sq0)
            pltpu.async_copy(k_hbm.at[src_v.at[b0 + 2]], kb0, sk0)

        pltpu.make_async_copy(p_hbm.at[dst_v.at[b0 + 1]], qb1, sq1).wait()
        pltpu.make_async_copy(k_hbm.at[src_v.at[b0 + 1]], kb1, sk1).wait()
        process(b0 + 1, qb1, kb1)
        return carry

    lax.fori_loop(0, RPT // 2, loop, 0)
    pltpu.sync_copy(ex_v, ex_hbm.at[wid])
    pltpu.sync_copy(den_v, den_hbm.at[pl.ds(wid * N, N)])


def _make_edge(D):
    return pl.kernel(
        functools.partial(_edge_body, D),
        out_type=[jax.ShapeDtypeStruct((NW, RPT, B), jnp.float32),
                  jax.ShapeDtypeStruct((NW * N,), jnp.float32)],
        mesh=_MESH,
        compiler_params=pltpu.CompilerParams(use_tc_tiling_on_sc=False, needs_layout_passes=False),
        scratch_types=[
            pltpu.VMEM((RPT, B), jnp.int32),
            pltpu.VMEM((RPT, B), jnp.int32),
            pltpu.VMEM((RPT, B), jnp.float32),
            pltpu.VMEM((N,), jnp.float32),
            pltpu.VMEM((B, D), jnp.float32),
            pltpu.VMEM((B, D), jnp.float32),
            pltpu.VMEM((B, D), jnp.float32),
            pltpu.VMEM((B, D), jnp.float32),
            pltpu.SemaphoreType.DMA,
            pltpu.SemaphoreType.DMA,
            pltpu.SemaphoreType.DMA,
            pltpu.SemaphoreType.DMA,
        ],
    )


def _agg_body(Dc, v_hbm, src_hbm, dst_hbm, ex_hbm, ds_hbm, z_hbm, o_hbm,
              src_v, dst_v, w_v, den_v, vb0, vb1, s0, s1, spm):
    c = lax.axis_index("c")
    s = lax.axis_index("s")
    wid = c * NS + s

    @pl.when(s == 0)
    def _():
        pltpu.sync_copy(z_hbm, spm)
    plsc.subcore_barrier()

    pltpu.sync_copy(ds_hbm.at[0], den_v)
    pltpu.sync_copy(src_hbm.at[wid], src_v)
    pltpu.sync_copy(dst_hbm.at[wid], dst_v)
    pltpu.sync_copy(ex_hbm.at[wid], w_v)

    def wstep(b, carry):
        for g in range(B // 16):
            d16 = dst_v[b, pl.ds(g * 16, 16)]
            dv = plsc.load_gather(den_v, [d16])
            w_v[b, pl.ds(g * 16, 16)] = \
                w_v[b, pl.ds(g * 16, 16)] / (dv + 1e-16)
        return carry

    lax.fori_loop(0, RPT, wstep, 0)

    def process(b, vb):
        brow = jnp.full((16,), b, jnp.int32)
        for g in range(B // 16):
            for e in range(16):
                col = jnp.full((16,), g * 16 + e, jnp.int32)
                sp = plsc.load_gather(w_v, [brow, col])
                for j in range(Dc // 16):
                    vb[g * 16 + e, pl.ds(j * 16, 16)] = \
                        vb[g * 16 + e, pl.ds(j * 16, 16)] * sp
        pltpu.sync_copy(vb, spm.at[dst_v.at[b]], add=True)

    pltpu.async_copy(v_hbm.at[src_v.at[0]], vb0, s0)

    def loop(i, carry):
        b0 = 2 * i
        pltpu.async_copy(v_hbm.at[src_v.at[b0 + 1]], vb1, s1)
        pltpu.make_async_copy(v_hbm.at[src_v.at[b0]], vb0, s0).wait()
        process(b0, vb0)

        @pl.when(i < RPT // 2 - 1)
        def _():
            pltpu.async_copy(v_hbm.at[src_v.at[b0 + 2]], vb0, s0)

        pltpu.make_async_copy(v_hbm.at[src_v.at[b0 + 1]], vb1, s1).wait()
        process(b0 + 1, vb1)
        return carry

    lax.fori_loop(0, RPT // 2, loop, 0)

    plsc.subcore_barrier()

    @pl.when(s == 0)
    def _():
        pltpu.sync_copy(spm, o_hbm.at[c])


def _make_agg(Dc):
    return pl.kernel(
        functools.partial(_agg_body, Dc),
        out_type=jax.ShapeDtypeStruct((NC, N, Dc), jnp.float32),
        mesh=_MESH,
        compiler_params=pltpu.CompilerParams(use_tc_tiling_on_sc=False, needs_layout_passes=False),
        scratch_types=[
            pltpu.VMEM((RPT, B), jnp.int32),
            pltpu.VMEM((RPT, B), jnp.int32),
            pltpu.VMEM((RPT, B), jnp.float32),
            pltpu.VMEM((N,), jnp.float32),
            pltpu.VMEM((B, Dc), jnp.float32),
            pltpu.VMEM((B, Dc), jnp.float32),
            pltpu.SemaphoreType.DMA,
            pltpu.SemaphoreType.DMA,
            pltpu.VMEM_SHARED((N, Dc), jnp.float32),
        ],
    )


_edge144 = _make_edge(D1P)
_edge64 = _make_edge(D2)
_agg64 = _make_agg(D2)


def kernel(x, edge_index, Wq1, bq1, Wk1, bk1, Wv1, bv1, Ws1, bs1,
           Wq2, bq2, Wk2, bk2, Wv2, bv2, Ws2, bs2):
    f32 = jnp.float32
    xp = jnp.concatenate(
        [x, jnp.ones((N, 1), f32), jnp.zeros((N, D1P - 129), f32)], axis=1)
    wq1p = jnp.concatenate(
        [Wq1, bq1[None, :], jnp.zeros((D1P - 129, D1), f32)], axis=0)
    wk1p = jnp.concatenate(
        [Wk1, bk1[None, :], jnp.zeros((D1P - 129, D1), f32)], axis=0)

    m1 = _mmt(wq1p, wk1p, 1.0 / (D1 ** 0.5))
    p1 = _rowmm(xp, m1)

    wvs1 = jnp.concatenate([Wv1, Ws1], axis=1)
    bvs1 = jnp.concatenate([bv1, bs1])[None, :]
    *v1c, s1 = _proj1(x, wvs1, bvs1)

    src2 = edge_index[0].reshape(NW, RPT, B)
    dst2 = edge_index[1].reshape(NW, RPT, B)
    z64 = jnp.zeros((N, D2), f32)

    ex1, den1 = _edge144(p1, xp, src2, dst2)
    ds1 = _densum(den1.reshape(NW, N))
    o1 = [_agg64(v, src2, dst2, ex1, ds1, z64) for v in v1c]

    w2 = jnp.concatenate([Wq2, Wk2, Wv2, Ws2], axis=1)
    b2 = jnp.concatenate([bq2, bk2, bv2, bs2])[None, :]
    q2s, k2, v2, s2 = _out1proj2(o1, s1, w2, b2)

    ex2, den2 = _edge64(q2s, k2, src2, dst2)
    ds2 = _densum(den2.reshape(NW, N))
    o2 = _agg64(v2, src2, dst2, ex2, ds2, z64)
    return _out2(o2, s2)


# speedup vs baseline: 5.8907x; 1.0030x over previous
import functools

import jax
import jax.numpy as jnp
from jax import lax
from jax.experimental import pallas as pl
from jax.experimental.pallas import tpu as pltpu
from jax.experimental.pallas import tpu_sc as plsc

N = 10000
E = 320000
NC = 2
NS = 16
NW = NC * NS
B = 80
RPT = E // NW // B
D1 = 512
D2 = 64
D1P = 144


def _elu(x):
    return jnp.where(x > 0, x, jnp.exp(jnp.minimum(x, 0.0)) - 1.0)


def _mmt_kernel(a_ref, b_ref, o_ref, *, scale):
    o_ref[...] = lax.dot_general(
        a_ref[...], b_ref[...], (((1,), (1,)), ((), ())),
        preferred_element_type=jnp.float32) * scale


def _mmt(a, b, scale):
    m, n = a.shape[0], b.shape[0]
    return pl.pallas_call(
        functools.partial(_mmt_kernel, scale=scale),
        out_shape=jax.ShapeDtypeStruct((m, n), jnp.float32),
    )(a, b)


def _rowmm_kernel(x_ref, w_ref, o_ref):
    o_ref[...] = jnp.dot(x_ref[...], w_ref[...],
                         preferred_element_type=jnp.float32)


def _rowmm(x, w, blk=1000):
    n, k = x.shape
    m = w.shape[1]
    return pl.pallas_call(
        _rowmm_kernel,
        grid=(n // blk,),
        in_specs=[pl.BlockSpec((blk, k), lambda i: (i, 0)),
                  pl.BlockSpec((k, m), lambda i: (0, 0))],
        out_specs=pl.BlockSpec((blk, m), lambda i: (i, 0)),
        out_shape=jax.ShapeDtypeStruct((n, m), jnp.float32),
    )(x, w)


def _proj1_kernel(x_ref, w_ref, b_ref, *out_refs):
    h = jnp.dot(x_ref[...], w_ref[...],
                preferred_element_type=jnp.float32) + b_ref[...]
    for i in range(8):
        out_refs[i][...] = h[:, 64 * i:64 * (i + 1)]
    out_refs[8][...] = h[:, 512:1024]


def _proj1(x, wvs, bvs, blk=1000):
    vshape = jax.ShapeDtypeStruct((N, 64), jnp.float32)
    return pl.pallas_call(
        _proj1_kernel,
        grid=(N // blk,),
        in_specs=[pl.BlockSpec((blk, 128), lambda i: (i, 0)),
                  pl.BlockSpec((128, 1024), lambda i: (0, 0)),
                  pl.BlockSpec((1, 1024), lambda i: (0, 0))],
        out_specs=[pl.BlockSpec((blk, 64), lambda i: (i, 0))] * 8
        + [pl.BlockSpec((blk, 512), lambda i: (i, 0))],
        out_shape=[vshape] * 8 + [jax.ShapeDtypeStruct((N, 512), jnp.float32)],
    )(x, wvs, bvs)


def _densum_kernel(d_ref, o_ref):
    o_ref[...] = jnp.sum(d_ref[...], axis=0, keepdims=True)


def _densum(den):
    return pl.pallas_call(
        _densum_kernel,
        out_shape=jax.ShapeDtypeStruct((1, N), jnp.float32),
    )(den)


def _out1proj2_kernel(*refs):
    ocs = refs[:8]
    s_ref, w_ref, b_ref, q2, k2, v2, s2 = refs[8:]
    h1 = jnp.concatenate(
        [o[0] + o[1] for o in ocs], axis=1) + s_ref[...]
    h1 = _elu(h1)
    hh = jnp.dot(h1, w_ref[...], preferred_element_type=jnp.float32) \
        + b_ref[...]
    q2[...] = hh[:, 0:64] * 0.125
    k2[...] = hh[:, 64:128]
    v2[...] = hh[:, 128:192]
    s2[...] = hh[:, 192:256]


def _out1proj2(oc, s1, w2, b2, blk=1000):
    oshape = jax.ShapeDtypeStruct((N, 64), jnp.float32)
    return pl.pallas_call(
        _out1proj2_kernel,
        grid=(N // blk,),
        in_specs=[pl.BlockSpec((NC, blk, 64), lambda i: (0, i, 0))] * 8
        + [pl.BlockSpec((blk, 512), lambda i: (i, 0)),
           pl.BlockSpec((512, 256), lambda i: (0, 0)),
           pl.BlockSpec((1, 256), lambda i: (0, 0))],
        out_specs=[pl.BlockSpec((blk, 64), lambda i: (i, 0))] * 4,
        out_shape=[oshape] * 4,
    )(*oc, s1, w2, b2)


def _out2_kernel(o_ref, s_ref, h_ref):
    h_ref[...] = _elu(o_ref[0] + o_ref[1] + s_ref[...])


def _out2(o2, s2, blk=1000):
    return pl.pallas_call(
        _out2_kernel,
        grid=(N // blk,),
        in_specs=[pl.BlockSpec((NC, blk, 64), lambda i: (0, i, 0)),
                  pl.BlockSpec((blk, 64), lambda i: (i, 0))],
        out_specs=pl.BlockSpec((blk, 64), lambda i: (i, 0)),
        out_shape=jax.ShapeDtypeStruct((N, 64), jnp.float32),
    )(o2, s2)


_MESH = plsc.VectorSubcoreMesh(core_axis_name="c", subcore_axis_name="s",
                               num_cores=NC, num_subcores=NS)


def _edge_body(D, p_hbm, k_hbm, src_hbm, dst_hbm, ex_hbm, den_hbm,
               src_v, dst_v, ex_v, den_v, qb0, kb0, qb1, kb1,
               sq0, sk0, sq1, sk1):
    c = lax.axis_index("c")
    s = lax.axis_index("s")
    wid = c * NS + s
    pltpu.sync_copy(src_hbm.at[wid], src_v)
    pltpu.sync_copy(dst_hbm.at[wid], dst_v)

    zero16 = jnp.zeros((16,), jnp.float32)

    def zden(i, carry):
        den_v[pl.ds(i * 16, 16)] = zero16
        return carry

    lax.fori_loop(0, N // 16, zden, 0)

    iota = lax.iota(jnp.int32, 16)

    def process(b, qb, kb):
        for g in range(B // 16):
            row16 = iota + g * 16
            d16 = dst_v[b, pl.ds(g * 16, 16)]

            def dotstep(j16, acc):
                for jj in range(16):
                    col = jnp.full((16,), j16 * 16 + jj, jnp.int32)
                    acc = acc + (plsc.load_gather(qb, [row16, col])
                                 * plsc.load_gather(kb, [row16, col]))
                return acc

            acc = lax.fori_loop(0, D // 16, dotstep, zero16)
            ex16 = jnp.exp(acc)
            ex_v[b, pl.ds(g * 16, 16)] = ex16
            plsc.addupdate_scatter(den_v, [d16], ex16)

    pltpu.async_copy(p_hbm.at[dst_v.at[0]], qb0, sq0)
    pltpu.async_copy(k_hbm.at[src_v.at[0]], kb0, sk0)

    def loop(i, carry):
        b0 = 2 * i
        pltpu.async_copy(p_hbm.at[dst_v.at[b0 + 1]], qb1, sq1)
        pltpu.async_copy(k_hbm.at[src_v.at[b0 + 1]], kb1, sk1)
        pltpu.make_async_copy(p_hbm.at[dst_v.at[b0]], qb0, sq0).wait()
        pltpu.make_async_copy(k_hbm.at[src_v.at[b0]], kb0, sk0).wait()
        process(b0, qb0, kb0)
        pltpu.async_copy(p_hbm.at[dst_v.at[b0 + 2]], qb0, sq0)
        pltpu.async_copy(k_hbm.at[src_v.at[b0 + 2]], kb0, sk0)
        pltpu.make_async_copy(p_hbm.at[dst_v.at[b0 + 1]], qb1, sq1).wait()
        pltpu.make_async_copy(k_hbm.at[src_v.at[b0 + 1]], kb1, sk1).wait()
        process(b0 + 1, qb1, kb1)
        return carry

    lax.fori_loop(0, RPT // 2, loop, 0)
    pltpu.make_async_copy(p_hbm.at[dst_v.at[RPT - 1]], qb0, sq0).wait()
    pltpu.make_async_copy(k_hbm.at[src_v.at[RPT - 1]], kb0, sk0).wait()
    process(RPT - 1, qb0, kb0)
    pltpu.sync_copy(ex_v, ex_hbm.at[wid])
    pltpu.sync_copy(den_v, den_hbm.at[pl.ds(wid * N, N)])


def _make_edge(D):
    return pl.kernel(
        functools.partial(_edge_body, D),
        out_type=[jax.ShapeDtypeStruct((NW, RPT, B), jnp.float32),
                  jax.ShapeDtypeStruct((NW * N,), jnp.float32)],
        mesh=_MESH,
        compiler_params=pltpu.CompilerParams(use_tc_tiling_on_sc=False, needs_layout_passes=False),
        scratch_types=[
            pltpu.VMEM((RPT, B), jnp.int32),
            pltpu.VMEM((RPT, B), jnp.int32),
            pltpu.VMEM((RPT, B), jnp.float32),
            pltpu.VMEM((N,), jnp.float32),
            pltpu.VMEM((B, D), jnp.float32),
            pltpu.VMEM((B, D), jnp.float32),
            pltpu.VMEM((B, D), jnp.float32),
            pltpu.VMEM((B, D), jnp.float32),
            pltpu.SemaphoreType.DMA,
            pltpu.SemaphoreType.DMA,
            pltpu.SemaphoreType.DMA,
            pltpu.SemaphoreType.DMA,
        ],
    )


def _agg_body(Dc, v_hbm, src_hbm, dst_hbm, ex_hbm, ds_hbm, z_hbm, o_hbm,
              src_v, dst_v, w_v, den_v, vb0, vb1, s0, s1, spm):
    c = lax.axis_index("c")
    s = lax.axis_index("s")
    wid = c * NS + s

    @pl.when(s == 0)
    def _():
        pltpu.sync_copy(z_hbm, spm)
    plsc.subcore_barrier()

    pltpu.sync_copy(ds_hbm.at[0], den_v)
    pltpu.sync_copy(src_hbm.at[wid], src_v)
    pltpu.sync_copy(dst_hbm.at[wid], dst_v)
    pltpu.sync_copy(ex_hbm.at[wid], w_v)

    def wstep(b, carry):
        for g in range(B // 16):
            d16 = dst_v[b, pl.ds(g * 16, 16)]
            dv = plsc.load_gather(den_v, [d16])
            w_v[b, pl.ds(g * 16, 16)] = \
                w_v[b, pl.ds(g * 16, 16)] / (dv + 1e-16)
        return carry

    lax.fori_loop(0, RPT, wstep, 0)

    def process(b, vb):
        brow = jnp.full((16,), b, jnp.int32)
        for g in range(B // 16):
            for e in range(16):
                col = jnp.full((16,), g * 16 + e, jnp.int32)
                sp = plsc.load_gather(w_v, [brow, col])
                for j in range(Dc // 16):
                    vb[g * 16 + e, pl.ds(j * 16, 16)] = \
                        vb[g * 16 + e, pl.ds(j * 16, 16)] * sp
        pltpu.sync_copy(vb, spm.at[dst_v.at[b]], add=True)

    pltpu.async_copy(v_hbm.at[src_v.at[0]], vb0, s0)

    def loop(i, carry):
        b0 = 2 * i
        pltpu.async_copy(v_hbm.at[src_v.at[b0 + 1]], vb1, s1)
        pltpu.make_async_copy(v_hbm.at[src_v.at[b0]], vb0, s0).wait()
        process(b0, vb0)
        pltpu.async_copy(v_hbm.at[src_v.at[b0 + 2]], vb0, s0)
        pltpu.make_async_copy(v_hbm.at[src_v.at[b0 + 1]], vb1, s1).wait()
        process(b0 + 1, vb1)
        return carry

    lax.fori_loop(0, RPT // 2, loop, 0)
    pltpu.make_async_copy(v_hbm.at[src_v.at[RPT - 1]], vb0, s0).wait()
    process(RPT - 1, vb0)

    plsc.subcore_barrier()

    @pl.when(s == 0)
    def _():
        pltpu.sync_copy(spm, o_hbm.at[c])


def _make_agg(Dc):
    return pl.kernel(
        functools.partial(_agg_body, Dc),
        out_type=jax.ShapeDtypeStruct((NC, N, Dc), jnp.float32),
        mesh=_MESH,
        compiler_params=pltpu.CompilerParams(use_tc_tiling_on_sc=False, needs_layout_passes=False),
        scratch_types=[
            pltpu.VMEM((RPT, B), jnp.int32),
            pltpu.VMEM((RPT, B), jnp.int32),
            pltpu.VMEM((RPT, B), jnp.float32),
            pltpu.VMEM((N,), jnp.float32),
            pltpu.VMEM((B, Dc), jnp.float32),
            pltpu.VMEM((B, Dc), jnp.float32),
            pltpu.SemaphoreType.DMA,
            pltpu.SemaphoreType.DMA,
            pltpu.VMEM_SHARED((N, Dc), jnp.float32),
        ],
    )


_edge144 = _make_edge(D1P)
_edge64 = _make_edge(D2)
_agg64 = _make_agg(D2)


def kernel(x, edge_index, Wq1, bq1, Wk1, bk1, Wv1, bv1, Ws1, bs1,
           Wq2, bq2, Wk2, bk2, Wv2, bv2, Ws2, bs2):
    f32 = jnp.float32
    xp = jnp.concatenate(
        [x, jnp.ones((N, 1), f32), jnp.zeros((N, D1P - 129), f32)], axis=1)
    wq1p = jnp.concatenate(
        [Wq1, bq1[None, :], jnp.zeros((D1P - 129, D1), f32)], axis=0)
    wk1p = jnp.concatenate(
        [Wk1, bk1[None, :], jnp.zeros((D1P - 129, D1), f32)], axis=0)

    m1 = _mmt(wq1p, wk1p, 1.0 / (D1 ** 0.5))
    p1 = _rowmm(xp, m1)

    wvs1 = jnp.concatenate([Wv1, Ws1], axis=1)
    bvs1 = jnp.concatenate([bv1, bs1])[None, :]
    *v1c, s1 = _proj1(x, wvs1, bvs1)

    src2 = edge_index[0].reshape(NW, RPT, B)
    dst2 = edge_index[1].reshape(NW, RPT, B)
    z64 = jnp.zeros((N, D2), f32)

    ex1, den1 = _edge144(p1, xp, src2, dst2)
    ds1 = _densum(den1.reshape(NW, N))
    o1 = [_agg64(v, src2, dst2, ex1, ds1, z64) for v in v1c]

    w2 = jnp.concatenate([Wq2, Wk2, Wv2, Ws2], axis=1)
    b2 = jnp.concatenate([bq2, bk2, bv2, bs2])[None, :]
    q2s, k2, v2, s2 = _out1proj2(o1, s1, w2, b2)

    ex2, den2 = _edge64(q2s, k2, src2, dst2)
    ds2 = _densum(den2.reshape(NW, N))
    o2 = _agg64(v2, src2, dst2, ex2, ds2, z64)
    return _out2(o2, s2)
